# baseline (speedup 1.0000x reference)
import functools

import jax
import jax.numpy as jnp
from jax import lax
from jax.experimental import pallas as pl
from jax.experimental.pallas import tpu as pltpu
from jax.experimental.pallas import tpu_sc as plsc

_LENGTH = 5
_EMBED = 768
_POOL = 2000
_TOPK = 5
_BATCH = 128
_SEQ = 196
_BCHUNK = 16

_HROW = _LENGTH * _EMBED
_NHALF = 2 * _POOL
_NW = 32
_CH = 5
_CROWS = 8


def _tc_body(x_ref, key_ref, sim_ref, idx_ref, rsum_ref):
    i = pl.program_id(0)
    x = x_ref[...]
    xm = jnp.mean(x, axis=1)
    xsq = jnp.sum(xm * xm, axis=-1, keepdims=True)
    xn = xm * lax.rsqrt(jnp.maximum(xsq, 1e-12))
    k = key_ref[...]
    ksq = jnp.sum(k * k, axis=-1, keepdims=True)
    kn = k * lax.rsqrt(jnp.maximum(ksq, 1e-12))
    sim = lax.dot_general(xn, kn, (((1,), (1,)), ((), ())),
                          preferred_element_type=jnp.float32)
    sim_ref[...] = sim

    col = lax.broadcasted_iota(jnp.int32, (_BCHUNK, _POOL), 1)
    work = sim
    vals_total = jnp.float32(0.0)
    half_idxs = []
    for _ in range(_TOPK):
        m = jnp.max(work, axis=1, keepdims=True)
        sel = jnp.where(work == m, col, _POOL)
        ik = jnp.min(sel, axis=1, keepdims=True)
        half_idxs.append(ik * 2)
        half_idxs.append(ik * 2 + 1)
        vals_total = vals_total + jnp.sum(m)
        work = jnp.where(col == ik, -jnp.inf, work)
    idx_ref[...] = jnp.concatenate(half_idxs, axis=1)

    @pl.when(i == 0)
    def _():
        rsum_ref[...] = jnp.zeros((8, 128), jnp.float32)

    rsum_ref[...] = rsum_ref[...] + jnp.full((8, 128), vals_total / _BATCH,
                                             jnp.float32)


def _tc_call(x_embed, prompt_key):
    return pl.pallas_call(
        _tc_body,
        grid=(_BATCH // _BCHUNK,),
        in_specs=[
            pl.BlockSpec((_BCHUNK, _SEQ, _EMBED), lambda i: (i, 0, 0)),
            pl.BlockSpec((_POOL, _EMBED), lambda i: (0, 0)),
        ],
        out_specs=[
            pl.BlockSpec((_BCHUNK, _POOL), lambda i: (i, 0)),
            pl.BlockSpec((_BCHUNK, 2 * _TOPK), lambda i: (i, 0)),
            pl.BlockSpec((8, 128), lambda i: (0, 0)),
        ],
        out_shape=[
            jax.ShapeDtypeStruct((_BATCH, _POOL), jnp.float32),
            jax.ShapeDtypeStruct((_BATCH, 2 * _TOPK), jnp.int32),
            jax.ShapeDtypeStruct((8, 128), jnp.float32),
        ],
    )(x_embed, prompt_key)


def _sc_body(table_hbm, idx_hbm, out_hbm, idx_v, buf0, buf1, sg0, sg1, so0, so1):
    wid = lax.axis_index("s") * 2 + lax.axis_index("c")
    pltpu.sync_copy(idx_hbm.at[wid], idx_v)

    def gather(c, buf, sem):
        return pltpu.async_copy(table_hbm.at[idx_v.at[c]], buf, sem)

    def put(c, buf, sem):
        return pltpu.async_copy(buf, out_hbm.at[wid, c], sem)

    g0 = gather(0, buf0, sg0)
    g1 = gather(1, buf1, sg1)
    g0.wait()
    o0 = put(0, buf0, so0)
    g1.wait()
    o1 = put(1, buf1, so1)
    o0.wait()
    g2 = gather(2, buf0, sg0)
    o1.wait()
    g3 = gather(3, buf1, sg1)
    g2.wait()
    o2 = put(2, buf0, so0)
    g3.wait()
    o3 = put(3, buf1, so1)
    o2.wait()
    g4 = gather(4, buf0, sg0)
    g4.wait()
    o4 = put(4, buf0, so0)
    o3.wait()
    o4.wait()


@functools.cache
def _sc_gather():
    return pl.kernel(
        _sc_body,
        mesh=plsc.VectorSubcoreMesh(core_axis_name="c", subcore_axis_name="s"),
        out_type=jax.ShapeDtypeStruct((_NW, _CH, _CROWS, _HROW), jnp.float32),
        scratch_types=[
            pltpu.VMEM((_CH, _CROWS), jnp.int32),
            pltpu.VMEM((_CROWS, _HROW), jnp.float32),
            pltpu.VMEM((_CROWS, _HROW), jnp.float32),
            pltpu.SemaphoreType.DMA,
            pltpu.SemaphoreType.DMA,
            pltpu.SemaphoreType.DMA,
            pltpu.SemaphoreType.DMA,
        ],
    )


def kernel(x_embed, e_p_0, prompt_key, layer_num=0):
    sim, idx2, rsum = _tc_call(x_embed, prompt_key)
    table_h = e_p_0.reshape(_NHALF, _HROW)
    idx_r = idx2.reshape(_NW, _CH, _CROWS)
    rows = _sc_gather()(table_h, idx_r)
    batched_prompt = rows.reshape(_BATCH, _TOPK * 2 * _LENGTH, _EMBED)
    reduce_sim = rsum[0, 0]
    return (batched_prompt, sim, reduce_sim)
